# baseline (speedup 1.0000x reference)
import jax
import jax.numpy as jnp
from jax.experimental import pallas as pl
from jax.experimental.pallas import tpu as pltpu


def _ru(a, m):
    return ((a + m - 1) // m) * m


_VMEM_LIMIT = 56 * 1024 * 1024


def _cp(sem):
    return pltpu.CompilerParams(
        dimension_semantics=tuple(sem),
        vmem_limit_bytes=_VMEM_LIMIT,
    )


_TILE_BUDGET = 40_000_000


def _conv_vmem(th, H, W, Wpad, ct, cins, pool):
    L = th * Wpad
    Lx = L + 2 * Wpad + 8
    nv = 6 if H // th > 1 else 2
    b = 0
    for cin in cins:
        b += nv * th * W * cin * 2
        b += 2 * (th + 3) * Wpad * cin * 2
        b += 6 * Lx * cin
        b += 2 * 9 * cin * ct * 2 * 2
    b += 3 * 4 * Lx * ct
    b += 2 * 2 * th * W * ct
    if pool:
        b += 2 * (th // 2) * (W // 2) * ct * 2
    return b


def _conv_geom(H, W, Cout, cins, pool):
    Wpad = _ru(W + 2, 8)
    ct = Cout if Cout <= 256 else 256
    th = 2
    for t in range(min(H, 64), 1, -1):
        if H % t == 0 and t % 2 == 0 and \
                _conv_vmem(t, H, W, Wpad, ct, cins, pool) <= _TILE_BUDGET:
            th = t
            break
    return Wpad, ct, th


def _make_conv_body(n_in, nv, th, W, Wpad, L, Lx, n_h, pool, head):
    def body(*refs):
        nx = n_in * nv
        x_refs = refs[:nx]
        w_refs = refs[nx:nx + n_in]
        b_ref = refs[nx + n_in]
        rest = refs[nx + n_in + 1:]
        if head:
            hw_ref, hb_ref = rest[0], rest[1]
            outs = rest[2:]
        else:
            outs = rest
        ct = b_ref.shape[1]
        h = pl.program_id(2)

        acc = jnp.zeros((L, ct), jnp.float32)
        for i in range(n_in):
            if nv == 3:
                pv = x_refs[3 * i][0]
                cu = x_refs[3 * i + 1][0]
                nx_ = x_refs[3 * i + 2][0]
                mt = (h > 0).astype(cu.dtype)
                mb = (h < n_h - 1).astype(cu.dtype)
                top = pv[th - 1:th] * mt
                bot = nx_[0:2] * mb
            else:
                cu = x_refs[i][0]
                C = cu.shape[-1]
                top = jnp.zeros((1, W, C), cu.dtype)
                bot = jnp.zeros((2, W, C), cu.dtype)
            win = jnp.concatenate([top, cu, bot], axis=0)
            C = win.shape[-1]
            zl = jnp.zeros((th + 3, 1, C), win.dtype)
            zr = jnp.zeros((th + 3, Wpad - W - 1, C), win.dtype)
            win = jnp.concatenate([zl, win, zr], axis=1)
            wf = win.reshape((th + 3) * Wpad, C)
            short = Lx + 2 - (th + 3) * Wpad
            if short > 0:
                wf = jnp.concatenate(
                    [wf, jnp.zeros((short, C), wf.dtype)], axis=0)
            x3 = jnp.concatenate(
                [wf[0:Lx], wf[1:Lx + 1], wf[2:Lx + 2]], axis=1)
            wk = w_refs[i]
            for dy in range(3):
                y = jnp.dot(x3, wk[dy], preferred_element_type=jnp.float32)
                s = dy * Wpad
                acc = acc + y[s:s + L]

        acc = jnp.maximum(acc + b_ref[...], 0.0)
        a3 = acc.reshape(th, Wpad, ct)[:, :W, :]
        if head:
            xb = a3.astype(jnp.bfloat16).astype(jnp.float32)
            hw = hw_ref[0].astype(jnp.float32)
            z = jnp.sum(xb * hw[None, None, :], axis=-1) + hb_ref[0, 0]
            outs[0][0] = jax.nn.sigmoid(z)
        else:
            ob = a3.astype(jnp.bfloat16)
            outs[0][0] = ob
            if pool:
                r5 = ob.reshape(th // 2, 2, W // 2, 2, ct)
                a = jnp.maximum(r5[:, 0], r5[:, 1])
                outs[1][0] = jnp.maximum(a[:, :, 0], a[:, :, 1])

    return body


def _conv3x3(xs, wks, b2, pool=False, head_wb=None):
    N, H, W, _ = xs[0].shape
    Cout = wks[0].shape[2]
    Wpad, ct, th = _conv_geom(H, W, Cout, [x.shape[3] for x in xs], pool)
    L = th * Wpad
    Lx = _ru(L + 2 * Wpad + 2, 8)
    n_h = H // th
    nv = 3 if n_h > 1 else 1
    nc = Cout // ct
    hmax = n_h - 1

    in_specs = []
    args = []
    for x in xs:
        C = x.shape[3]
        if nv == 3:
            in_specs += [
                pl.BlockSpec((1, th, W, C),
                             lambda n, c, h: (n, jnp.maximum(h - 1, 0), 0, 0)),
                pl.BlockSpec((1, th, W, C), lambda n, c, h: (n, h, 0, 0)),
                pl.BlockSpec((1, th, W, C),
                             lambda n, c, h: (n, jnp.minimum(h + 1, hmax), 0, 0)),
            ]
            args += [x, x, x]
        else:
            in_specs.append(
                pl.BlockSpec((1, th, W, C), lambda n, c, h: (n, h, 0, 0)))
            args.append(x)
    for wk in wks:
        cin = wk.shape[1]
        in_specs.append(
            pl.BlockSpec((3, 3 * cin, ct), lambda n, c, h: (0, 0, c)))
        args.append(wk.reshape(3, 3 * cin, Cout))
    in_specs.append(pl.BlockSpec((1, ct), lambda n, c, h: (0, c)))
    args.append(b2)

    head = head_wb is not None
    if head:
        hw, hb = head_wb
        in_specs.append(pl.BlockSpec((1, ct), lambda n, c, h: (0, 0)))
        in_specs.append(pl.BlockSpec((1, 1), lambda n, c, h: (0, 0)))
        args += [hw, hb]
        out_shape = jax.ShapeDtypeStruct((N, H, W), jnp.float32)
        out_specs = pl.BlockSpec((1, th, W), lambda n, c, h: (n, h, 0))
    elif pool:
        out_shape = (
            jax.ShapeDtypeStruct((N, H, W, Cout), jnp.bfloat16),
            jax.ShapeDtypeStruct((N, H // 2, W // 2, Cout), jnp.bfloat16),
        )
        out_specs = (
            pl.BlockSpec((1, th, W, ct), lambda n, c, h: (n, h, 0, c)),
            pl.BlockSpec((1, th // 2, W // 2, ct), lambda n, c, h: (n, h, 0, c)),
        )
    else:
        out_shape = jax.ShapeDtypeStruct((N, H, W, Cout), jnp.bfloat16)
        out_specs = pl.BlockSpec((1, th, W, ct), lambda n, c, h: (n, h, 0, c))

    return pl.pallas_call(
        _make_conv_body(len(xs), nv, th, W, Wpad, L, Lx, n_h, pool, head),
        out_shape=out_shape,
        grid=(N, nc, n_h),
        in_specs=in_specs,
        out_specs=out_specs,
        compiler_params=_cp(("parallel", "parallel", "arbitrary")),
    )(*args)


def _make_chain_body(n_conv, H, W, Wpad, L, Lx):
    def body(*refs):
        cur = refs[0][0]
        w_refs = refs[1:1 + n_conv]
        b_refs = refs[1 + n_conv:1 + 2 * n_conv]
        o_ref = refs[1 + 2 * n_conv]
        for j in range(n_conv):
            C = cur.shape[-1]
            win = jnp.concatenate(
                [jnp.zeros((1, W, C), cur.dtype), cur,
                 jnp.zeros((2, W, C), cur.dtype)], axis=0)
            win = jnp.concatenate(
                [jnp.zeros((H + 3, 1, C), win.dtype), win,
                 jnp.zeros((H + 3, Wpad - W - 1, C), win.dtype)], axis=1)
            wf = win.reshape((H + 3) * Wpad, C)
            short = Lx + 2 - (H + 3) * Wpad
            if short > 0:
                wf = jnp.concatenate(
                    [wf, jnp.zeros((short, C), wf.dtype)], axis=0)
            x3 = jnp.concatenate(
                [wf[0:Lx], wf[1:Lx + 1], wf[2:Lx + 2]], axis=1)
            acc = jnp.zeros((L, b_refs[j].shape[1]), jnp.float32)
            for dy in range(3):
                y = jnp.dot(x3, w_refs[j][dy],
                            preferred_element_type=jnp.float32)
                acc = acc + y[dy * Wpad:dy * Wpad + L]
            acc = jnp.maximum(acc + b_refs[j][...], 0.0)
            cur = acc.reshape(H, Wpad, -1)[:, :W, :].astype(jnp.bfloat16)
        o_ref[0] = cur

    return body


def _conv_chain(x, wks, b2s):
    N, H, W, _ = x.shape
    Wpad = _ru(W + 2, 8)
    L = H * Wpad
    Lx = _ru(L + 2 * Wpad + 2, 8)
    n_conv = len(wks)
    Cout = wks[-1].shape[2]
    in_specs = [pl.BlockSpec((1, H, W, x.shape[3]), lambda n: (n, 0, 0, 0))]
    args = [x]
    for wk in wks:
        cin = wk.shape[1]
        in_specs.append(
            pl.BlockSpec((3, 3 * cin, wk.shape[2]), lambda n: (0, 0, 0)))
        args.append(wk.reshape(3, 3 * cin, wk.shape[2]))
    for b2 in b2s:
        in_specs.append(pl.BlockSpec(b2.shape, lambda n: (0, 0)))
        args.append(b2)
    return pl.pallas_call(
        _make_chain_body(n_conv, H, W, Wpad, L, Lx),
        out_shape=jax.ShapeDtypeStruct((N, H, W, Cout), jnp.bfloat16),
        grid=(N,),
        in_specs=in_specs,
        out_specs=pl.BlockSpec((1, H, W, Cout), lambda n: (n, 0, 0, 0)),
        compiler_params=_cp(("parallel",)),
    )(*args)


def _make_ups_body(tu, W, Co):
    def body(x_ref, w_ref, o_ref):
        Cin = x_ref.shape[3]
        xf = x_ref[0].reshape(tu * W, Cin)
        y = jnp.dot(xf, w_ref[...],
                    preferred_element_type=jnp.float32).astype(jnp.bfloat16)
        y = y.reshape(tu, W, 2, 2, Co).transpose(0, 2, 1, 3, 4)
        o_ref[0] = y.reshape(2 * tu, 2 * W, Co)

    return body


def _upsample2x(x, wk):
    N, H, W, Cin = x.shape
    C4 = wk.shape[1]
    Co = C4 // 4
    tu = 1
    for t in range(H, 0, -1):
        if H % t == 0 and t * W <= 4096:
            tu = t
            break
    return pl.pallas_call(
        _make_ups_body(tu, W, Co),
        out_shape=jax.ShapeDtypeStruct((N, 2 * H, 2 * W, Co), jnp.bfloat16),
        grid=(N, H // tu),
        in_specs=[
            pl.BlockSpec((1, tu, W, Cin), lambda n, h: (n, h, 0, 0)),
            pl.BlockSpec((Cin, C4), lambda n, h: (0, 0)),
        ],
        out_specs=pl.BlockSpec((1, 2 * tu, 2 * W, Co), lambda n, h: (n, h, 0, 0)),
        compiler_params=_cp(("parallel", "arbitrary")),
    )(x, wk)


def kernel(conv1_1_w, conv1_1_b, conv1_2_w, conv1_2_b, conv2_1_w, conv2_1_b,
           conv2_2_w, conv2_2_b, conv3_1_w, conv3_1_b, conv3_2_w, conv3_2_b,
           conv3_3_w, conv3_3_b, conv4_1_w, conv4_1_b, conv4_2_w, conv4_2_b,
           conv4_3_w, conv4_3_b, conv5_1_w, conv5_1_b, conv5_2_w, conv5_2_b,
           conv5_3_w, conv5_3_b, conv6_1_w, conv6_1_b, conv6_2_w, conv6_2_b,
           conv6_3_w, conv6_3_b, conv7_1_wa, conv7_1_wb, conv7_1_b, conv7_2_w,
           conv7_2_b, conv7_3_w, conv7_3_b, conv8_1_wa, conv8_1_wb, conv8_1_b,
           conv8_2_w, conv8_2_b, conv8_3_w, conv8_3_b, conv9_1_wa, conv9_1_wb,
           conv9_1_b, conv9_2_w, conv9_2_b, conv10_1_wa, conv10_1_wb,
           conv10_1_b, conv10_2_w, conv10_2_b, up6_w, up7_w, up8_w, up9_w,
           output_w, output_b, x):
    t = jnp.transpose(x, (0, 2, 3, 1)).astype(jnp.bfloat16)

    t = _conv3x3([t], [conv1_1_w], conv1_1_b)
    f1, t = _conv3x3([t], [conv1_2_w], conv1_2_b, pool=True)
    t = _conv3x3([t], [conv2_1_w], conv2_1_b)
    f2, t = _conv3x3([t], [conv2_2_w], conv2_2_b, pool=True)
    t = _conv_chain(t, [conv3_1_w, conv3_2_w], [conv3_1_b, conv3_2_b])
    f3, t = _conv3x3([t], [conv3_3_w], conv3_3_b, pool=True)
    t = _conv_chain(t, [conv4_1_w, conv4_2_w], [conv4_1_b, conv4_2_b])
    f4, t = _conv3x3([t], [conv4_3_w], conv4_3_b, pool=True)
    t = _conv_chain(t, [conv5_1_w, conv5_2_w, conv5_3_w],
                    [conv5_1_b, conv5_2_b, conv5_3_b])
    t = _conv_chain(t, [conv6_1_w, conv6_2_w, conv6_3_w],
                    [conv6_1_b, conv6_2_b, conv6_3_b])

    t = _upsample2x(t, up6_w)
    t = _conv3x3([f4, t], [conv7_1_wa, conv7_1_wb], conv7_1_b)
    t = _conv_chain(t, [conv7_2_w, conv7_3_w], [conv7_2_b, conv7_3_b])

    t = _upsample2x(t, up7_w)
    t = _conv3x3([f3, t], [conv8_1_wa, conv8_1_wb], conv8_1_b)
    t = _conv_chain(t, [conv8_2_w, conv8_3_w], [conv8_2_b, conv8_3_b])

    t = _upsample2x(t, up8_w)
    t = _conv3x3([f2, t], [conv9_1_wa, conv9_1_wb], conv9_1_b)
    t = _conv3x3([t], [conv9_2_w], conv9_2_b)

    t = _upsample2x(t, up9_w)
    t = _conv3x3([f1, t], [conv10_1_wa, conv10_1_wb], conv10_1_b)

    hw = output_w.reshape(1, 64)
    y = _conv3x3([t], [conv10_2_w], conv10_2_b, head_wb=(hw, output_b))
    return y[:, None, :, :]


# speedup vs baseline: 1.0234x; 1.0234x over previous
import jax
import jax.numpy as jnp
from jax.experimental import pallas as pl
from jax.experimental.pallas import tpu as pltpu


def _ru(a, m):
    return ((a + m - 1) // m) * m


_VMEM_LIMIT = 56 * 1024 * 1024


def _cp(sem):
    return pltpu.CompilerParams(
        dimension_semantics=tuple(sem),
        vmem_limit_bytes=_VMEM_LIMIT,
    )


_TILE_BUDGET = 49_000_000


def _conv_vmem(th, H, W, Wpad, ct, cins, pool):
    L = th * Wpad
    Lx = L + 2 * Wpad + 8
    nv = 6 if H // th > 1 else 2
    b = 0
    for cin in cins:
        b += nv * th * W * cin * 2
        b += 2 * (th + 3) * Wpad * cin * 2
        b += 6 * Lx * cin
        b += 2 * 9 * cin * ct * 2 * 2
    b += 3 * 4 * Lx * ct
    b += 2 * 2 * th * W * ct
    if pool:
        b += 2 * (th // 2) * (W // 2) * ct * 2
    return b


def _conv_geom(H, W, Cout, cins, pool):
    Wpad = _ru(W + 2, 8)
    ct = Cout if Cout <= 256 else 256
    th = 2
    for t in range(min(H, 64), 1, -1):
        if H % t == 0 and t % 2 == 0 and \
                _conv_vmem(t, H, W, Wpad, ct, cins, pool) <= _TILE_BUDGET:
            th = t
            break
    return Wpad, ct, th


def _make_conv_body(n_in, nv, th, W, Wpad, L, Lx, n_h, pool, head):
    def body(*refs):
        nx = n_in * nv
        x_refs = refs[:nx]
        w_refs = refs[nx:nx + n_in]
        b_ref = refs[nx + n_in]
        rest = refs[nx + n_in + 1:]
        if head:
            hw_ref, hb_ref = rest[0], rest[1]
            outs = rest[2:]
        else:
            outs = rest
        ct = b_ref.shape[1]
        h = pl.program_id(2)

        acc = jnp.zeros((L, ct), jnp.float32)
        for i in range(n_in):
            if nv == 3:
                pv = x_refs[3 * i][0]
                cu = x_refs[3 * i + 1][0]
                nx_ = x_refs[3 * i + 2][0]
                mt = (h > 0).astype(cu.dtype)
                mb = (h < n_h - 1).astype(cu.dtype)
                top = pv[th - 1:th] * mt
                bot = nx_[0:2] * mb
            else:
                cu = x_refs[i][0]
                C = cu.shape[-1]
                top = jnp.zeros((1, W, C), cu.dtype)
                bot = jnp.zeros((2, W, C), cu.dtype)
            win = jnp.concatenate([top, cu, bot], axis=0)
            C = win.shape[-1]
            zl = jnp.zeros((th + 3, 1, C), win.dtype)
            zr = jnp.zeros((th + 3, Wpad - W - 1, C), win.dtype)
            win = jnp.concatenate([zl, win, zr], axis=1)
            wf = win.reshape((th + 3) * Wpad, C)
            short = Lx + 2 - (th + 3) * Wpad
            if short > 0:
                wf = jnp.concatenate(
                    [wf, jnp.zeros((short, C), wf.dtype)], axis=0)
            x3 = jnp.concatenate(
                [wf[0:Lx], wf[1:Lx + 1], wf[2:Lx + 2]], axis=1)
            wk = w_refs[i]
            for dy in range(3):
                y = jnp.dot(x3, wk[dy], preferred_element_type=jnp.float32)
                s = dy * Wpad
                acc = acc + y[s:s + L]

        acc = jnp.maximum(acc + b_ref[...], 0.0)
        a3 = acc.reshape(th, Wpad, ct)[:, :W, :]
        if head:
            xb = a3.astype(jnp.bfloat16).astype(jnp.float32)
            hw = hw_ref[0].astype(jnp.float32)
            z = jnp.sum(xb * hw[None, None, :], axis=-1) + hb_ref[0, 0]
            outs[0][0] = jax.nn.sigmoid(z)
        else:
            ob = a3.astype(jnp.bfloat16)
            outs[0][0] = ob
            if pool:
                r5 = ob.reshape(th // 2, 2, W // 2, 2, ct)
                a = jnp.maximum(r5[:, 0], r5[:, 1])
                outs[1][0] = jnp.maximum(a[:, :, 0], a[:, :, 1])

    return body


def _conv3x3(xs, wks, b2, pool=False, head_wb=None):
    N, H, W, _ = xs[0].shape
    Cout = wks[0].shape[2]
    Wpad, ct, th = _conv_geom(H, W, Cout, [x.shape[3] for x in xs], pool)
    L = th * Wpad
    Lx = _ru(L + 2 * Wpad + 2, 8)
    n_h = H // th
    nv = 3 if n_h > 1 else 1
    nc = Cout // ct
    hmax = n_h - 1

    in_specs = []
    args = []
    for x in xs:
        C = x.shape[3]
        if nv == 3:
            in_specs += [
                pl.BlockSpec((1, th, W, C),
                             lambda n, c, h: (n, jnp.maximum(h - 1, 0), 0, 0)),
                pl.BlockSpec((1, th, W, C), lambda n, c, h: (n, h, 0, 0)),
                pl.BlockSpec((1, th, W, C),
                             lambda n, c, h: (n, jnp.minimum(h + 1, hmax), 0, 0)),
            ]
            args += [x, x, x]
        else:
            in_specs.append(
                pl.BlockSpec((1, th, W, C), lambda n, c, h: (n, h, 0, 0)))
            args.append(x)
    for wk in wks:
        cin = wk.shape[1]
        in_specs.append(
            pl.BlockSpec((3, 3 * cin, ct), lambda n, c, h: (0, 0, c)))
        args.append(wk.reshape(3, 3 * cin, Cout))
    in_specs.append(pl.BlockSpec((1, ct), lambda n, c, h: (0, c)))
    args.append(b2)

    head = head_wb is not None
    if head:
        hw, hb = head_wb
        in_specs.append(pl.BlockSpec((1, ct), lambda n, c, h: (0, 0)))
        in_specs.append(pl.BlockSpec((1, 1), lambda n, c, h: (0, 0)))
        args += [hw, hb]
        out_shape = jax.ShapeDtypeStruct((N, H, W), jnp.float32)
        out_specs = pl.BlockSpec((1, th, W), lambda n, c, h: (n, h, 0))
    elif pool:
        out_shape = (
            jax.ShapeDtypeStruct((N, H, W, Cout), jnp.bfloat16),
            jax.ShapeDtypeStruct((N, H // 2, W // 2, Cout), jnp.bfloat16),
        )
        out_specs = (
            pl.BlockSpec((1, th, W, ct), lambda n, c, h: (n, h, 0, c)),
            pl.BlockSpec((1, th // 2, W // 2, ct), lambda n, c, h: (n, h, 0, c)),
        )
    else:
        out_shape = jax.ShapeDtypeStruct((N, H, W, Cout), jnp.bfloat16)
        out_specs = pl.BlockSpec((1, th, W, ct), lambda n, c, h: (n, h, 0, c))

    return pl.pallas_call(
        _make_conv_body(len(xs), nv, th, W, Wpad, L, Lx, n_h, pool, head),
        out_shape=out_shape,
        grid=(N, nc, n_h),
        in_specs=in_specs,
        out_specs=out_specs,
        compiler_params=_cp(("parallel", "parallel", "arbitrary")),
    )(*args)


def _make_chain_body(n_conv, H, W, Wpad, L, Lx):
    def body(*refs):
        cur = refs[0][0]
        w_refs = refs[1:1 + n_conv]
        b_refs = refs[1 + n_conv:1 + 2 * n_conv]
        o_ref = refs[1 + 2 * n_conv]
        for j in range(n_conv):
            C = cur.shape[-1]
            win = jnp.concatenate(
                [jnp.zeros((1, W, C), cur.dtype), cur,
                 jnp.zeros((2, W, C), cur.dtype)], axis=0)
            win = jnp.concatenate(
                [jnp.zeros((H + 3, 1, C), win.dtype), win,
                 jnp.zeros((H + 3, Wpad - W - 1, C), win.dtype)], axis=1)
            wf = win.reshape((H + 3) * Wpad, C)
            short = Lx + 2 - (H + 3) * Wpad
            if short > 0:
                wf = jnp.concatenate(
                    [wf, jnp.zeros((short, C), wf.dtype)], axis=0)
            x3 = jnp.concatenate(
                [wf[0:Lx], wf[1:Lx + 1], wf[2:Lx + 2]], axis=1)
            acc = jnp.zeros((L, b_refs[j].shape[1]), jnp.float32)
            for dy in range(3):
                y = jnp.dot(x3, w_refs[j][dy],
                            preferred_element_type=jnp.float32)
                acc = acc + y[dy * Wpad:dy * Wpad + L]
            acc = jnp.maximum(acc + b_refs[j][...], 0.0)
            cur = acc.reshape(H, Wpad, -1)[:, :W, :].astype(jnp.bfloat16)
        o_ref[0] = cur

    return body


def _conv_chain(x, wks, b2s):
    N, H, W, _ = x.shape
    Wpad = _ru(W + 2, 8)
    L = H * Wpad
    Lx = _ru(L + 2 * Wpad + 2, 8)
    n_conv = len(wks)
    Cout = wks[-1].shape[2]
    in_specs = [pl.BlockSpec((1, H, W, x.shape[3]), lambda n: (n, 0, 0, 0))]
    args = [x]
    for wk in wks:
        cin = wk.shape[1]
        in_specs.append(
            pl.BlockSpec((3, 3 * cin, wk.shape[2]), lambda n: (0, 0, 0)))
        args.append(wk.reshape(3, 3 * cin, wk.shape[2]))
    for b2 in b2s:
        in_specs.append(pl.BlockSpec(b2.shape, lambda n: (0, 0)))
        args.append(b2)
    return pl.pallas_call(
        _make_chain_body(n_conv, H, W, Wpad, L, Lx),
        out_shape=jax.ShapeDtypeStruct((N, H, W, Cout), jnp.bfloat16),
        grid=(N,),
        in_specs=in_specs,
        out_specs=pl.BlockSpec((1, H, W, Cout), lambda n: (n, 0, 0, 0)),
        compiler_params=_cp(("parallel",)),
    )(*args)


def _make_ups_body(tu, W, Co):
    def body(x_ref, w_ref, o_ref):
        Cin = x_ref.shape[3]
        xf = x_ref[0].reshape(tu * W, Cin)
        y = jnp.dot(xf, w_ref[...],
                    preferred_element_type=jnp.float32).astype(jnp.bfloat16)
        y = y.reshape(tu, W, 2, 2, Co).transpose(0, 2, 1, 3, 4)
        o_ref[0] = y.reshape(2 * tu, 2 * W, Co)

    return body


def _upsample2x(x, wk):
    N, H, W, Cin = x.shape
    C4 = wk.shape[1]
    Co = C4 // 4
    tu = 1
    for t in range(H, 0, -1):
        if H % t == 0 and t * W <= 4096:
            tu = t
            break
    return pl.pallas_call(
        _make_ups_body(tu, W, Co),
        out_shape=jax.ShapeDtypeStruct((N, 2 * H, 2 * W, Co), jnp.bfloat16),
        grid=(N, H // tu),
        in_specs=[
            pl.BlockSpec((1, tu, W, Cin), lambda n, h: (n, h, 0, 0)),
            pl.BlockSpec((Cin, C4), lambda n, h: (0, 0)),
        ],
        out_specs=pl.BlockSpec((1, 2 * tu, 2 * W, Co), lambda n, h: (n, h, 0, 0)),
        compiler_params=_cp(("parallel", "arbitrary")),
    )(x, wk)


def kernel(conv1_1_w, conv1_1_b, conv1_2_w, conv1_2_b, conv2_1_w, conv2_1_b,
           conv2_2_w, conv2_2_b, conv3_1_w, conv3_1_b, conv3_2_w, conv3_2_b,
           conv3_3_w, conv3_3_b, conv4_1_w, conv4_1_b, conv4_2_w, conv4_2_b,
           conv4_3_w, conv4_3_b, conv5_1_w, conv5_1_b, conv5_2_w, conv5_2_b,
           conv5_3_w, conv5_3_b, conv6_1_w, conv6_1_b, conv6_2_w, conv6_2_b,
           conv6_3_w, conv6_3_b, conv7_1_wa, conv7_1_wb, conv7_1_b, conv7_2_w,
           conv7_2_b, conv7_3_w, conv7_3_b, conv8_1_wa, conv8_1_wb, conv8_1_b,
           conv8_2_w, conv8_2_b, conv8_3_w, conv8_3_b, conv9_1_wa, conv9_1_wb,
           conv9_1_b, conv9_2_w, conv9_2_b, conv10_1_wa, conv10_1_wb,
           conv10_1_b, conv10_2_w, conv10_2_b, up6_w, up7_w, up8_w, up9_w,
           output_w, output_b, x):
    t = jnp.transpose(x, (0, 2, 3, 1)).astype(jnp.bfloat16)

    t = _conv3x3([t], [conv1_1_w], conv1_1_b)
    f1, t = _conv3x3([t], [conv1_2_w], conv1_2_b, pool=True)
    t = _conv3x3([t], [conv2_1_w], conv2_1_b)
    f2, t = _conv3x3([t], [conv2_2_w], conv2_2_b, pool=True)
    t = _conv3x3([t], [conv3_1_w], conv3_1_b)
    t = _conv3x3([t], [conv3_2_w], conv3_2_b)
    f3, t = _conv3x3([t], [conv3_3_w], conv3_3_b, pool=True)
    t = _conv3x3([t], [conv4_1_w], conv4_1_b)
    t = _conv3x3([t], [conv4_2_w], conv4_2_b)
    f4, t = _conv3x3([t], [conv4_3_w], conv4_3_b, pool=True)
    t = _conv_chain(t, [conv5_1_w, conv5_2_w, conv5_3_w],
                    [conv5_1_b, conv5_2_b, conv5_3_b])
    t = _conv_chain(t, [conv6_1_w, conv6_2_w, conv6_3_w],
                    [conv6_1_b, conv6_2_b, conv6_3_b])

    t = _upsample2x(t, up6_w)
    t = _conv3x3([f4, t], [conv7_1_wa, conv7_1_wb], conv7_1_b)
    t = _conv3x3([t], [conv7_2_w], conv7_2_b)
    t = _conv3x3([t], [conv7_3_w], conv7_3_b)

    t = _upsample2x(t, up7_w)
    t = _conv3x3([f3, t], [conv8_1_wa, conv8_1_wb], conv8_1_b)
    t = _conv3x3([t], [conv8_2_w], conv8_2_b)
    t = _conv3x3([t], [conv8_3_w], conv8_3_b)

    t = _upsample2x(t, up8_w)
    t = _conv3x3([f2, t], [conv9_1_wa, conv9_1_wb], conv9_1_b)
    t = _conv3x3([t], [conv9_2_w], conv9_2_b)

    t = _upsample2x(t, up9_w)
    t = _conv3x3([f1, t], [conv10_1_wa, conv10_1_wb], conv10_1_b)

    hw = output_w.reshape(1, 64)
    y = _conv3x3([t], [conv10_2_w], conv10_2_b, head_wb=(hw, output_b))
    return y[:, None, :, :]


# speedup vs baseline: 1.0239x; 1.0005x over previous
import jax
import jax.numpy as jnp
from jax.experimental import pallas as pl
from jax.experimental.pallas import tpu as pltpu


def _ru(a, m):
    return ((a + m - 1) // m) * m


_VMEM_LIMIT = 56 * 1024 * 1024


def _cp(sem):
    return pltpu.CompilerParams(
        dimension_semantics=tuple(sem),
        vmem_limit_bytes=_VMEM_LIMIT,
    )


_TILE_BUDGET = 51_000_000


def _conv_vmem(th, H, W, Wpad, ct, cins, pool):
    L = th * Wpad
    Lx = L + 2 * Wpad + 8
    nv = 6 if H // th > 1 else 2
    b = 0
    for cin in cins:
        b += nv * th * W * cin * 2
        b += 2 * (th + 3) * Wpad * cin * 2
        b += 6 * Lx * cin
        b += 2 * 9 * cin * ct * 2 * 2
    b += 3 * 4 * Lx * ct
    b += 2 * 2 * th * W * ct
    if pool:
        b += 2 * (th // 2) * (W // 2) * ct * 2
    return b


def _conv_geom(H, W, Cout, cins, pool):
    Wpad = _ru(W + 2, 8)
    ct = Cout if Cout <= 256 else 256
    th = 2
    for t in range(min(H, 64), 1, -1):
        if H % t == 0 and t % 2 == 0 and \
                _conv_vmem(t, H, W, Wpad, ct, cins, pool) <= _TILE_BUDGET:
            th = t
            break
    return Wpad, ct, th


def _make_conv_body(n_in, nv, th, W, Wpad, L, Lx, n_h, pool, head):
    def body(*refs):
        nx = n_in * nv
        x_refs = refs[:nx]
        w_refs = refs[nx:nx + n_in]
        b_ref = refs[nx + n_in]
        rest = refs[nx + n_in + 1:]
        if head:
            hw_ref, hb_ref = rest[0], rest[1]
            outs = rest[2:]
        else:
            outs = rest
        ct = b_ref.shape[1]
        h = pl.program_id(2)

        acc = jnp.zeros((L, ct), jnp.float32)
        for i in range(n_in):
            if nv == 3:
                pv = x_refs[3 * i][0]
                cu = x_refs[3 * i + 1][0]
                nx_ = x_refs[3 * i + 2][0]
                mt = (h > 0).astype(cu.dtype)
                mb = (h < n_h - 1).astype(cu.dtype)
                top = pv[th - 1:th] * mt
                bot = nx_[0:2] * mb
            else:
                cu = x_refs[i][0]
                C = cu.shape[-1]
                top = jnp.zeros((1, W, C), cu.dtype)
                bot = jnp.zeros((2, W, C), cu.dtype)
            win = jnp.concatenate([top, cu, bot], axis=0)
            C = win.shape[-1]
            zl = jnp.zeros((th + 3, 1, C), win.dtype)
            zr = jnp.zeros((th + 3, Wpad - W - 1, C), win.dtype)
            win = jnp.concatenate([zl, win, zr], axis=1)
            wf = win.reshape((th + 3) * Wpad, C)
            short = Lx + 2 - (th + 3) * Wpad
            if short > 0:
                wf = jnp.concatenate(
                    [wf, jnp.zeros((short, C), wf.dtype)], axis=0)
            x3 = jnp.concatenate(
                [wf[0:Lx], wf[1:Lx + 1], wf[2:Lx + 2]], axis=1)
            wk = w_refs[i]
            for dy in range(3):
                y = jnp.dot(x3, wk[dy], preferred_element_type=jnp.float32)
                s = dy * Wpad
                acc = acc + y[s:s + L]

        acc = jnp.maximum(acc + b_ref[...], 0.0)
        a3 = acc.reshape(th, Wpad, ct)[:, :W, :]
        if head:
            xb = a3.astype(jnp.bfloat16).astype(jnp.float32)
            hw = hw_ref[0].astype(jnp.float32)
            z = jnp.sum(xb * hw[None, None, :], axis=-1) + hb_ref[0, 0]
            outs[0][0] = jax.nn.sigmoid(z)
        else:
            ob = a3.astype(jnp.bfloat16)
            outs[0][0] = ob
            if pool:
                r5 = ob.reshape(th // 2, 2, W // 2, 2, ct)
                a = jnp.maximum(r5[:, 0], r5[:, 1])
                outs[1][0] = jnp.maximum(a[:, :, 0], a[:, :, 1])

    return body


def _conv3x3(xs, wks, b2, pool=False, head_wb=None):
    N, H, W, _ = xs[0].shape
    Cout = wks[0].shape[2]
    Wpad, ct, th = _conv_geom(H, W, Cout, [x.shape[3] for x in xs], pool)
    L = th * Wpad
    Lx = _ru(L + 2 * Wpad + 2, 8)
    n_h = H // th
    nv = 3 if n_h > 1 else 1
    nc = Cout // ct
    hmax = n_h - 1

    in_specs = []
    args = []
    for x in xs:
        C = x.shape[3]
        if nv == 3:
            in_specs += [
                pl.BlockSpec((1, th, W, C),
                             lambda n, c, h: (n, jnp.maximum(h - 1, 0), 0, 0)),
                pl.BlockSpec((1, th, W, C), lambda n, c, h: (n, h, 0, 0)),
                pl.BlockSpec((1, th, W, C),
                             lambda n, c, h: (n, jnp.minimum(h + 1, hmax), 0, 0)),
            ]
            args += [x, x, x]
        else:
            in_specs.append(
                pl.BlockSpec((1, th, W, C), lambda n, c, h: (n, h, 0, 0)))
            args.append(x)
    for wk in wks:
        cin = wk.shape[1]
        in_specs.append(
            pl.BlockSpec((3, 3 * cin, ct), lambda n, c, h: (0, 0, c)))
        args.append(wk.reshape(3, 3 * cin, Cout))
    in_specs.append(pl.BlockSpec((1, ct), lambda n, c, h: (0, c)))
    args.append(b2)

    head = head_wb is not None
    if head:
        hw, hb = head_wb
        in_specs.append(pl.BlockSpec((1, ct), lambda n, c, h: (0, 0)))
        in_specs.append(pl.BlockSpec((1, 1), lambda n, c, h: (0, 0)))
        args += [hw, hb]
        out_shape = jax.ShapeDtypeStruct((N, H, W), jnp.float32)
        out_specs = pl.BlockSpec((1, th, W), lambda n, c, h: (n, h, 0))
    elif pool:
        out_shape = (
            jax.ShapeDtypeStruct((N, H, W, Cout), jnp.bfloat16),
            jax.ShapeDtypeStruct((N, H // 2, W // 2, Cout), jnp.bfloat16),
        )
        out_specs = (
            pl.BlockSpec((1, th, W, ct), lambda n, c, h: (n, h, 0, c)),
            pl.BlockSpec((1, th // 2, W // 2, ct), lambda n, c, h: (n, h, 0, c)),
        )
    else:
        out_shape = jax.ShapeDtypeStruct((N, H, W, Cout), jnp.bfloat16)
        out_specs = pl.BlockSpec((1, th, W, ct), lambda n, c, h: (n, h, 0, c))

    return pl.pallas_call(
        _make_conv_body(len(xs), nv, th, W, Wpad, L, Lx, n_h, pool, head),
        out_shape=out_shape,
        grid=(N, nc, n_h),
        in_specs=in_specs,
        out_specs=out_specs,
        compiler_params=_cp(("parallel", "parallel", "arbitrary")),
    )(*args)


def _make_chain_body(n_conv, H, W, Wpad, L, Lx):
    def body(*refs):
        cur = refs[0][0]
        w_refs = refs[1:1 + n_conv]
        b_refs = refs[1 + n_conv:1 + 2 * n_conv]
        o_ref = refs[1 + 2 * n_conv]
        for j in range(n_conv):
            C = cur.shape[-1]
            win = jnp.concatenate(
                [jnp.zeros((1, W, C), cur.dtype), cur,
                 jnp.zeros((2, W, C), cur.dtype)], axis=0)
            win = jnp.concatenate(
                [jnp.zeros((H + 3, 1, C), win.dtype), win,
                 jnp.zeros((H + 3, Wpad - W - 1, C), win.dtype)], axis=1)
            wf = win.reshape((H + 3) * Wpad, C)
            short = Lx + 2 - (H + 3) * Wpad
            if short > 0:
                wf = jnp.concatenate(
                    [wf, jnp.zeros((short, C), wf.dtype)], axis=0)
            x3 = jnp.concatenate(
                [wf[0:Lx], wf[1:Lx + 1], wf[2:Lx + 2]], axis=1)
            acc = jnp.zeros((L, b_refs[j].shape[1]), jnp.float32)
            for dy in range(3):
                y = jnp.dot(x3, w_refs[j][dy],
                            preferred_element_type=jnp.float32)
                acc = acc + y[dy * Wpad:dy * Wpad + L]
            acc = jnp.maximum(acc + b_refs[j][...], 0.0)
            cur = acc.reshape(H, Wpad, -1)[:, :W, :].astype(jnp.bfloat16)
        o_ref[0] = cur

    return body


def _conv_chain(x, wks, b2s):
    N, H, W, _ = x.shape
    Wpad = _ru(W + 2, 8)
    L = H * Wpad
    Lx = _ru(L + 2 * Wpad + 2, 8)
    n_conv = len(wks)
    Cout = wks[-1].shape[2]
    in_specs = [pl.BlockSpec((1, H, W, x.shape[3]), lambda n: (n, 0, 0, 0))]
    args = [x]
    for wk in wks:
        cin = wk.shape[1]
        in_specs.append(
            pl.BlockSpec((3, 3 * cin, wk.shape[2]), lambda n: (0, 0, 0)))
        args.append(wk.reshape(3, 3 * cin, wk.shape[2]))
    for b2 in b2s:
        in_specs.append(pl.BlockSpec(b2.shape, lambda n: (0, 0)))
        args.append(b2)
    return pl.pallas_call(
        _make_chain_body(n_conv, H, W, Wpad, L, Lx),
        out_shape=jax.ShapeDtypeStruct((N, H, W, Cout), jnp.bfloat16),
        grid=(N,),
        in_specs=in_specs,
        out_specs=pl.BlockSpec((1, H, W, Cout), lambda n: (n, 0, 0, 0)),
        compiler_params=_cp(("parallel",)),
    )(*args)


def _make_ups_body(tu, W, Co):
    def body(x_ref, w_ref, o_ref):
        Cin = x_ref.shape[3]
        xf = x_ref[0].reshape(tu * W, Cin)
        y = jnp.dot(xf, w_ref[...],
                    preferred_element_type=jnp.float32).astype(jnp.bfloat16)
        y = y.reshape(tu, W, 2, 2, Co).transpose(0, 2, 1, 3, 4)
        o_ref[0] = y.reshape(2 * tu, 2 * W, Co)

    return body


def _upsample2x(x, wk):
    N, H, W, Cin = x.shape
    C4 = wk.shape[1]
    Co = C4 // 4
    tu = 1
    for t in range(H, 0, -1):
        if H % t == 0 and t * W <= 4096:
            tu = t
            break
    return pl.pallas_call(
        _make_ups_body(tu, W, Co),
        out_shape=jax.ShapeDtypeStruct((N, 2 * H, 2 * W, Co), jnp.bfloat16),
        grid=(N, H // tu),
        in_specs=[
            pl.BlockSpec((1, tu, W, Cin), lambda n, h: (n, h, 0, 0)),
            pl.BlockSpec((Cin, C4), lambda n, h: (0, 0)),
        ],
        out_specs=pl.BlockSpec((1, 2 * tu, 2 * W, Co), lambda n, h: (n, h, 0, 0)),
        compiler_params=_cp(("parallel", "arbitrary")),
    )(x, wk)


def kernel(conv1_1_w, conv1_1_b, conv1_2_w, conv1_2_b, conv2_1_w, conv2_1_b,
           conv2_2_w, conv2_2_b, conv3_1_w, conv3_1_b, conv3_2_w, conv3_2_b,
           conv3_3_w, conv3_3_b, conv4_1_w, conv4_1_b, conv4_2_w, conv4_2_b,
           conv4_3_w, conv4_3_b, conv5_1_w, conv5_1_b, conv5_2_w, conv5_2_b,
           conv5_3_w, conv5_3_b, conv6_1_w, conv6_1_b, conv6_2_w, conv6_2_b,
           conv6_3_w, conv6_3_b, conv7_1_wa, conv7_1_wb, conv7_1_b, conv7_2_w,
           conv7_2_b, conv7_3_w, conv7_3_b, conv8_1_wa, conv8_1_wb, conv8_1_b,
           conv8_2_w, conv8_2_b, conv8_3_w, conv8_3_b, conv9_1_wa, conv9_1_wb,
           conv9_1_b, conv9_2_w, conv9_2_b, conv10_1_wa, conv10_1_wb,
           conv10_1_b, conv10_2_w, conv10_2_b, up6_w, up7_w, up8_w, up9_w,
           output_w, output_b, x):
    t = jnp.transpose(x, (0, 2, 3, 1)).astype(jnp.bfloat16)

    t = _conv3x3([t], [conv1_1_w], conv1_1_b)
    f1, t = _conv3x3([t], [conv1_2_w], conv1_2_b, pool=True)
    t = _conv3x3([t], [conv2_1_w], conv2_1_b)
    f2, t = _conv3x3([t], [conv2_2_w], conv2_2_b, pool=True)
    t = _conv3x3([t], [conv3_1_w], conv3_1_b)
    t = _conv3x3([t], [conv3_2_w], conv3_2_b)
    f3, t = _conv3x3([t], [conv3_3_w], conv3_3_b, pool=True)
    t = _conv3x3([t], [conv4_1_w], conv4_1_b)
    t = _conv3x3([t], [conv4_2_w], conv4_2_b)
    f4, t = _conv3x3([t], [conv4_3_w], conv4_3_b, pool=True)
    t = _conv_chain(t, [conv5_1_w, conv5_2_w, conv5_3_w],
                    [conv5_1_b, conv5_2_b, conv5_3_b])
    t = _conv_chain(t, [conv6_1_w, conv6_2_w, conv6_3_w],
                    [conv6_1_b, conv6_2_b, conv6_3_b])

    t = _upsample2x(t, up6_w)
    t = _conv3x3([f4, t], [conv7_1_wa, conv7_1_wb], conv7_1_b)
    t = _conv3x3([t], [conv7_2_w], conv7_2_b)
    t = _conv3x3([t], [conv7_3_w], conv7_3_b)

    t = _upsample2x(t, up7_w)
    t = _conv3x3([f3, t], [conv8_1_wa, conv8_1_wb], conv8_1_b)
    t = _conv3x3([t], [conv8_2_w], conv8_2_b)
    t = _conv3x3([t], [conv8_3_w], conv8_3_b)

    t = _upsample2x(t, up8_w)
    t = _conv3x3([f2, t], [conv9_1_wa, conv9_1_wb], conv9_1_b)
    t = _conv3x3([t], [conv9_2_w], conv9_2_b)

    t = _upsample2x(t, up9_w)
    t = _conv3x3([f1, t], [conv10_1_wa, conv10_1_wb], conv10_1_b)

    hw = output_w.reshape(1, 64)
    y = _conv3x3([t], [conv10_2_w], conv10_2_b, head_wb=(hw, output_b))
    return y[:, None, :, :]
